# Initial kernel scaffold; baseline (speedup 1.0000x reference)
import jax
import jax.numpy as jnp
from jax.experimental import pallas as pl


def kernel(x, hyperedge_index, W1, b1, W2, b2, W3, b3, W4, b4, W5, b5, fc1_w, fc1_b, fc2_w, fc2_b, fco_w, fco_b):
    raise NotImplementedError("write your pallas kernel here")


# speedup vs baseline: 5.3092x; 5.3092x over previous
import functools

import jax
import jax.numpy as jnp
from jax import lax
from jax.experimental import pallas as pl
from jax.experimental.pallas import tpu as pltpu
from jax.experimental.pallas import tpu_sc as plsc

N = 10000
NP = 10240
E = 320000
EP = 327680
PER_TILE = EP // 16
NITER = 160
CW = 128
RPT = NP // 16

_MESH = plsc.VectorSubcoreMesh(core_axis_name="c", subcore_axis_name="s")


def _zero_rows(buf, nrows, ncols16):
    z = jnp.zeros((16,), jnp.float32)

    def body(r, _):
        for j in range(ncols16):
            buf[r, pl.ds(j * 16, 16)] = z
        return 0

    lax.fori_loop(0, nrows, body, 0)


def _degree_kernel(srcp, dstp):
    def body(src_hbm, dst_hbm, dinv_hbm, binv_hbm, idxv, ones_v, buf, acc, sem):
        c = lax.axis_index("c")
        s = lax.axis_index("s")
        for i in range(CW // 16):
            ones_v[pl.ds(i * 16, 16)] = jnp.ones((16,), jnp.float32)

        def bzero(i, _):
            buf[pl.ds(i * 16, 16)] = jnp.zeros((16,), jnp.float32)
            return 0

        lax.fori_loop(0, RPT // 16, bzero, 0)
        pltpu.sync_copy(buf, acc.at[pl.ds(s * RPT, RPT)])

        @pl.when(c == 0)
        def _():
            pltpu.sync_copy(src_hbm.at[s], idxv)

        @pl.when(c == 1)
        def _():
            pltpu.sync_copy(dst_hbm.at[s], idxv)

        plsc.subcore_barrier()

        def scat(j, _):
            pltpu.sync_copy(ones_v, acc.at[idxv.at[j]], add=True)
            return 0

        lax.fori_loop(0, NITER, scat, 0)
        plsc.subcore_barrier()

        pltpu.sync_copy(acc.at[pl.ds(s * RPT, RPT)], buf)

        def inv(i, _):
            v = buf[pl.ds(i * 16, 16)]
            buf[pl.ds(i * 16, 16)] = jnp.where(v > 0.0, 1.0 / v, 0.0)
            return 0

        lax.fori_loop(0, RPT // 16, inv, 0)

        @pl.when(c == 0)
        def _():
            pltpu.sync_copy(buf, dinv_hbm.at[pl.ds(s * RPT, RPT)])

        @pl.when(c == 1)
        def _():
            pltpu.sync_copy(buf, binv_hbm.at[pl.ds(s * RPT, RPT)])

    f = pl.kernel(
        body,
        out_type=(
            jax.ShapeDtypeStruct((NP,), jnp.float32),
            jax.ShapeDtypeStruct((NP,), jnp.float32),
        ),
        mesh=_MESH,
        scratch_types=(
            pltpu.VMEM((NITER, CW), jnp.int32),
            pltpu.VMEM((CW,), jnp.float32),
            pltpu.VMEM((RPT,), jnp.float32),
            pltpu.VMEM_SHARED((NP,), jnp.float32),
            pltpu.SemaphoreType.DMA,
        ),
    )
    return f(srcp, dstp)


def _make_hconv(nchunks, dc, fuse_bias_elu):
    nc_per_sc = nchunks // 2
    c16 = dc // 16

    def body(f_hbm, src_hbm, dst_hbm, binv_hbm, dinv_hbm, bias_hbm, o_hbm,
             srcv, dstv, rows, rbuf, svec, bvec, acc_e, acc_o, sem):
        c = lax.axis_index("c")
        s = lax.axis_index("s")
        pltpu.sync_copy(src_hbm.at[s], srcv)
        pltpu.sync_copy(dst_hbm.at[s], dstv)

        def do_chunk(k):
            _zero_rows(rbuf, RPT, c16)
            pltpu.sync_copy(rbuf, acc_e.at[pl.ds(s * RPT, RPT)])
            pltpu.sync_copy(rbuf, acc_o.at[pl.ds(s * RPT, RPT)])
            plsc.subcore_barrier()

            def p1(j, _):
                pltpu.async_copy(f_hbm.at[k].at[srcv.at[j]], rows, sem).wait()
                pltpu.sync_copy(rows, acc_e.at[dstv.at[j]], add=True)
                return 0

            lax.fori_loop(0, NITER, p1, 0)
            plsc.subcore_barrier()

            pltpu.sync_copy(binv_hbm.at[pl.ds(s * RPT, RPT)], svec)
            pltpu.sync_copy(acc_e.at[pl.ds(s * RPT, RPT)], rbuf)

            def srow(g, _):
                sv = svec[pl.ds(g * 16, 16)]
                for rr in range(16):
                    b = sv[rr]
                    r = g * 16 + rr
                    for j in range(c16):
                        rbuf[r, pl.ds(j * 16, 16)] = (
                            rbuf[r, pl.ds(j * 16, 16)] * b)
                return 0

            lax.fori_loop(0, RPT // 16, srow, 0)
            pltpu.sync_copy(rbuf, acc_e.at[pl.ds(s * RPT, RPT)])
            plsc.subcore_barrier()

            def p2(j, _):
                pltpu.async_copy(acc_e.at[dstv.at[j]], rows, sem).wait()
                pltpu.sync_copy(rows, acc_o.at[srcv.at[j]], add=True)
                return 0

            lax.fori_loop(0, NITER, p2, 0)
            plsc.subcore_barrier()

            pltpu.sync_copy(dinv_hbm.at[pl.ds(s * RPT, RPT)], svec)
            pltpu.sync_copy(acc_o.at[pl.ds(s * RPT, RPT)], rbuf)
            if fuse_bias_elu:
                pltpu.sync_copy(bias_hbm.at[pl.ds(k * dc, dc)], bvec)

            def erow(g, _):
                sv = svec[pl.ds(g * 16, 16)]
                for rr in range(16):
                    d = sv[rr]
                    r = g * 16 + rr
                    for j in range(c16):
                        v = rbuf[r, pl.ds(j * 16, 16)] * d
                        if fuse_bias_elu:
                            v = v + bvec[pl.ds(j * 16, 16)]
                            v = jnp.where(v > 0.0, v, jnp.exp(v) - 1.0)
                        rbuf[r, pl.ds(j * 16, 16)] = v
                return 0

            lax.fori_loop(0, RPT // 16, erow, 0)
            pltpu.sync_copy(rbuf, o_hbm.at[k].at[pl.ds(s * RPT, RPT)])
            plsc.subcore_barrier()

        @pl.when(c == 0)
        def _():
            for k in range(nc_per_sc):
                do_chunk(k)

        @pl.when(c == 1)
        def _():
            for k in range(nc_per_sc, nchunks):
                do_chunk(k)

    f = pl.kernel(
        body,
        out_type=jax.ShapeDtypeStruct((nchunks, NP, dc), jnp.float32),
        mesh=_MESH,
        scratch_types=(
            pltpu.VMEM((NITER, CW), jnp.int32),
            pltpu.VMEM((NITER, CW), jnp.int32),
            pltpu.VMEM((CW, dc), jnp.float32),
            pltpu.VMEM((RPT, dc), jnp.float32),
            pltpu.VMEM((RPT,), jnp.float32),
            pltpu.VMEM((dc,), jnp.float32),
            pltpu.VMEM_SHARED((NP, dc), jnp.float32),
            pltpu.VMEM_SHARED((NP, dc), jnp.float32),
            pltpu.SemaphoreType.DMA,
        ),
        compiler_params=pltpu.CompilerParams(use_tc_tiling_on_sc=False),
    )
    return f


_hconv128 = _make_hconv(4, 32, False)
_hconv256 = _make_hconv(8, 32, True)
_hconv64 = _make_hconv(2, 32, True)


def _dotT(x, w):
    return lax.dot_general(x, w, (((1,), (1,)), ((), ())),
                           preferred_element_type=jnp.float32)


RT = 1280


def _mm(xc, w, bias, act, ncout, dco):
    ncin, _, dci = xc.shape
    dout = w.shape[0]

    def body(x_ref, w_ref, b_ref, o_ref):
        wm = w_ref[...]
        acc = jnp.zeros((RT, dout), jnp.float32)
        for k in range(ncin):
            acc = acc + _dotT(x_ref[k], wm[:, k * dci:(k + 1) * dci])
        if b_ref is not None:
            acc = acc + b_ref[...]
        if act == "elu":
            acc = jnp.where(acc > 0.0, acc, jnp.exp(acc) - 1.0)
        for k in range(ncout):
            o_ref[k] = acc[:, k * dco:(k + 1) * dco]

    in_specs = [
        pl.BlockSpec((ncin, RT, dci), lambda i: (0, i, 0)),
        pl.BlockSpec((dout, w.shape[1]), lambda i: (0, 0)),
    ]
    args = [xc, w]
    if bias is not None:
        in_specs.append(pl.BlockSpec((1, dout), lambda i: (0, 0)))
        args.append(bias.reshape(1, dout))
        fn = body
    else:
        fn = lambda x_ref, w_ref, o_ref: body(x_ref, w_ref, None, o_ref)

    return pl.pallas_call(
        fn,
        grid=(NP // RT,),
        in_specs=in_specs,
        out_specs=pl.BlockSpec((ncout, RT, dco), lambda i: (0, i, 0)),
        out_shape=jax.ShapeDtypeStruct((ncout, NP, dco), jnp.float32),
    )(*args)


def _head(h, fc1_w, fc1_b, fc2_w, fc2_b, fco_w, fco_b):
    def body(x_ref, w1_ref, b1_ref, w2_ref, b2_ref, wo_ref, bo_ref, o_ref):
        x = jnp.concatenate([x_ref[0], x_ref[1]], axis=1)
        a = jnp.maximum(_dotT(x, w1_ref[...]) + b1_ref[...], 0.0)
        a = jnp.maximum(_dotT(a, w2_ref[...]) + b2_ref[...], 0.0)
        lg = _dotT(a, wo_ref[...]) + bo_ref[...]
        m = jnp.max(lg, axis=1, keepdims=True)
        e = jnp.exp(lg - m)
        o_ref[...] = e / jnp.sum(e, axis=1, keepdims=True)

    return pl.pallas_call(
        body,
        grid=(NP // RT,),
        in_specs=[
            pl.BlockSpec((2, RT, 32), lambda i: (0, i, 0)),
            pl.BlockSpec((64, 64), lambda i: (0, 0)),
            pl.BlockSpec((1, 64), lambda i: (0, 0)),
            pl.BlockSpec((64, 64), lambda i: (0, 0)),
            pl.BlockSpec((1, 64), lambda i: (0, 0)),
            pl.BlockSpec((2, 64), lambda i: (0, 0)),
            pl.BlockSpec((1, 2), lambda i: (0, 0)),
        ],
        out_specs=pl.BlockSpec((RT, 2), lambda i: (i, 0)),
        out_shape=jax.ShapeDtypeStruct((NP, 2), jnp.float32),
    )(h, fc1_w, fc1_b.reshape(1, 64), fc2_w, fc2_b.reshape(1, 64),
      fco_w, fco_b.reshape(1, 2))


def kernel(x, hyperedge_index, W1, b1, W2, b2, W3, b3, W4, b4, W5, b5,
           fc1_w, fc1_b, fc2_w, fc2_b, fco_w, fco_b):
    src = hyperedge_index[0]
    dst = hyperedge_index[1]
    npad = EP - E
    pad_idx = (jnp.arange(npad, dtype=jnp.int32) % (NP - N)) + N
    srcp = jnp.concatenate([src, pad_idx]).reshape(16, NITER, CW)
    dstp = jnp.concatenate([dst, pad_idx]).reshape(16, NITER, CW)

    dinv, binv = _degree_kernel(srcp, dstp)

    xp = jnp.pad(x, ((0, NP - N), (0, 0)))
    xc = jnp.stack([xp[:, i * 32:(i + 1) * 32] for i in range(4)])
    dummy = jnp.zeros((1,), jnp.float32)
    m1 = _hconv128(xc, srcp, dstp, binv, dinv, dummy)
    h = _mm(m1, W1, b1, "elu", 8, 32)

    for W, b in ((W2, b2), (W3, b3), (W4, b4)):
        t = _mm(h, W, None, None, 8, 32)
        h = _hconv256(t, srcp, dstp, binv, dinv, b)

    t5 = _mm(h, W5, None, None, 2, 32)
    h5 = _hconv64(t5, srcp, dstp, binv, dinv, b5)

    out = _head(h5, fc1_w, fc1_b, fc2_w, fc2_b, fco_w, fco_b)
    return out[:N]


# speedup vs baseline: 8.0487x; 1.5160x over previous
import functools

import jax
import jax.numpy as jnp
from jax import lax
from jax.experimental import pallas as pl
from jax.experimental.pallas import tpu as pltpu
from jax.experimental.pallas import tpu_sc as plsc

N = 10000
NP = 10240
E = 320000
EP = 327680
PER_TILE = EP // 16
NITER = 160
CW = 128
RPT = NP // 16

_MESH = plsc.VectorSubcoreMesh(core_axis_name="c", subcore_axis_name="s")


def _zero_rows(buf, nrows, ncols16):
    z = jnp.zeros((16,), jnp.float32)

    def body(r, _):
        for j in range(ncols16):
            buf[r, pl.ds(j * 16, 16)] = z
        return 0

    lax.fori_loop(0, nrows, body, 0)


def _degree_kernel(srcp, dstp):
    def body(src_hbm, dst_hbm, dinv_hbm, binv_hbm, idxv, ones_v, buf, acc, sem):
        c = lax.axis_index("c")
        s = lax.axis_index("s")
        for i in range(CW // 16):
            ones_v[pl.ds(i * 16, 16)] = jnp.ones((16,), jnp.float32)

        def bzero(i, _):
            buf[pl.ds(i * 16, 16)] = jnp.zeros((16,), jnp.float32)
            return 0

        lax.fori_loop(0, RPT // 16, bzero, 0)
        pltpu.sync_copy(buf, acc.at[pl.ds(s * RPT, RPT)])

        @pl.when(c == 0)
        def _():
            pltpu.sync_copy(src_hbm.at[s], idxv)

        @pl.when(c == 1)
        def _():
            pltpu.sync_copy(dst_hbm.at[s], idxv)

        plsc.subcore_barrier()

        def scat(j, _):
            pltpu.sync_copy(ones_v, acc.at[idxv.at[j]], add=True)
            return 0

        lax.fori_loop(0, NITER, scat, 0)
        plsc.subcore_barrier()

        pltpu.sync_copy(acc.at[pl.ds(s * RPT, RPT)], buf)

        def inv(i, _):
            v = buf[pl.ds(i * 16, 16)]
            buf[pl.ds(i * 16, 16)] = jnp.where(v > 0.0, 1.0 / v, 0.0)
            return 0

        lax.fori_loop(0, RPT // 16, inv, 0)

        @pl.when(c == 0)
        def _():
            pltpu.sync_copy(buf, dinv_hbm.at[pl.ds(s * RPT, RPT)])

        @pl.when(c == 1)
        def _():
            pltpu.sync_copy(buf, binv_hbm.at[pl.ds(s * RPT, RPT)])

    f = pl.kernel(
        body,
        out_type=(
            jax.ShapeDtypeStruct((NP,), jnp.float32),
            jax.ShapeDtypeStruct((NP,), jnp.float32),
        ),
        mesh=_MESH,
        scratch_types=(
            pltpu.VMEM((NITER, CW), jnp.int32),
            pltpu.VMEM((CW,), jnp.float32),
            pltpu.VMEM((RPT,), jnp.float32),
            pltpu.VMEM_SHARED((NP,), jnp.float32),
            pltpu.SemaphoreType.DMA,
        ),
    )
    return f(srcp, dstp)


def _make_hconv(nchunks, dc, fuse_bias_elu):
    nc_per_sc = nchunks // 2
    c16 = dc // 16

    def body(f_hbm, src_hbm, dst_hbm, binv_hbm, dinv_hbm, bias_hbm, o_hbm,
             srcv, dstv, rows0, rows1, rbuf, svec, bvec, acc_e, acc_o,
             sem0, sem1):
        c = lax.axis_index("c")
        s = lax.axis_index("s")
        pltpu.sync_copy(src_hbm.at[s], srcv)
        pltpu.sync_copy(dst_hbm.at[s], dstv)

        def pipelined_pass(gather_src, gidx, sidx, acc):
            def gref(j):
                return gather_src.at[gidx.at[j]]

            pltpu.async_copy(gref(0), rows0, sem0)

            def it(g, _):
                j0 = 2 * g
                j1 = 2 * g + 1
                d1 = pltpu.async_copy(gref(j1), rows1, sem1)
                pltpu.make_async_copy(gref(j0), rows0, sem0).wait()
                pltpu.sync_copy(rows0, acc.at[sidx.at[j0]], add=True)

                @pl.when(g + 1 < NITER // 2)
                def _():
                    pltpu.async_copy(gref(j0 + 2), rows0, sem0)

                d1.wait()
                pltpu.sync_copy(rows1, acc.at[sidx.at[j1]], add=True)
                return 0

            lax.fori_loop(0, NITER // 2, it, 0)

        def do_chunk(k):
            _zero_rows(rbuf, RPT, c16)
            pltpu.sync_copy(rbuf, acc_e.at[pl.ds(s * RPT, RPT)])
            pltpu.sync_copy(rbuf, acc_o.at[pl.ds(s * RPT, RPT)])
            plsc.subcore_barrier()

            pipelined_pass(f_hbm.at[k], srcv, dstv, acc_e)
            plsc.subcore_barrier()

            pltpu.sync_copy(binv_hbm.at[pl.ds(s * RPT, RPT)], svec)
            pltpu.sync_copy(acc_e.at[pl.ds(s * RPT, RPT)], rbuf)

            def srow(g, _):
                sv = svec[pl.ds(g * 16, 16)]
                for rr in range(16):
                    b = sv[rr]
                    r = g * 16 + rr
                    for j in range(c16):
                        rbuf[r, pl.ds(j * 16, 16)] = (
                            rbuf[r, pl.ds(j * 16, 16)] * b)
                return 0

            lax.fori_loop(0, RPT // 16, srow, 0)
            pltpu.sync_copy(rbuf, acc_e.at[pl.ds(s * RPT, RPT)])
            plsc.subcore_barrier()

            pipelined_pass(acc_e, dstv, srcv, acc_o)
            plsc.subcore_barrier()

            pltpu.sync_copy(dinv_hbm.at[pl.ds(s * RPT, RPT)], svec)
            pltpu.sync_copy(acc_o.at[pl.ds(s * RPT, RPT)], rbuf)
            if fuse_bias_elu:
                pltpu.sync_copy(bias_hbm.at[pl.ds(k * dc, dc)], bvec)

            def erow(g, _):
                sv = svec[pl.ds(g * 16, 16)]
                for rr in range(16):
                    d = sv[rr]
                    r = g * 16 + rr
                    for j in range(c16):
                        v = rbuf[r, pl.ds(j * 16, 16)] * d
                        if fuse_bias_elu:
                            v = v + bvec[pl.ds(j * 16, 16)]
                            v = jnp.where(v > 0.0, v, jnp.exp(v) - 1.0)
                        rbuf[r, pl.ds(j * 16, 16)] = v
                return 0

            lax.fori_loop(0, RPT // 16, erow, 0)
            pltpu.sync_copy(rbuf, o_hbm.at[k].at[pl.ds(s * RPT, RPT)])
            plsc.subcore_barrier()

        @pl.when(c == 0)
        def _():
            for k in range(nc_per_sc):
                do_chunk(k)

        @pl.when(c == 1)
        def _():
            for k in range(nc_per_sc, nchunks):
                do_chunk(k)

    f = pl.kernel(
        body,
        out_type=jax.ShapeDtypeStruct((nchunks, NP, dc), jnp.float32),
        mesh=_MESH,
        scratch_types=(
            pltpu.VMEM((NITER, CW), jnp.int32),
            pltpu.VMEM((NITER, CW), jnp.int32),
            pltpu.VMEM((CW, dc), jnp.float32),
            pltpu.VMEM((CW, dc), jnp.float32),
            pltpu.VMEM((RPT, dc), jnp.float32),
            pltpu.VMEM((RPT,), jnp.float32),
            pltpu.VMEM((dc,), jnp.float32),
            pltpu.VMEM_SHARED((NP, dc), jnp.float32),
            pltpu.VMEM_SHARED((NP, dc), jnp.float32),
            pltpu.SemaphoreType.DMA,
            pltpu.SemaphoreType.DMA,
        ),
        compiler_params=pltpu.CompilerParams(use_tc_tiling_on_sc=False),
    )
    return f


_hconv128 = _make_hconv(4, 32, False)
_hconv256 = _make_hconv(8, 32, True)
_hconv64 = _make_hconv(2, 32, True)


def _dotT(x, w):
    return lax.dot_general(x, w, (((1,), (1,)), ((), ())),
                           preferred_element_type=jnp.float32)


RT = 1280


def _mm(xc, w, bias, act, ncout, dco):
    ncin, _, dci = xc.shape
    dout = w.shape[0]

    def body(x_ref, w_ref, b_ref, o_ref):
        wm = w_ref[...]
        acc = jnp.zeros((RT, dout), jnp.float32)
        for k in range(ncin):
            acc = acc + _dotT(x_ref[k], wm[:, k * dci:(k + 1) * dci])
        if b_ref is not None:
            acc = acc + b_ref[...]
        if act == "elu":
            acc = jnp.where(acc > 0.0, acc, jnp.exp(acc) - 1.0)
        for k in range(ncout):
            o_ref[k] = acc[:, k * dco:(k + 1) * dco]

    in_specs = [
        pl.BlockSpec((ncin, RT, dci), lambda i: (0, i, 0)),
        pl.BlockSpec((dout, w.shape[1]), lambda i: (0, 0)),
    ]
    args = [xc, w]
    if bias is not None:
        in_specs.append(pl.BlockSpec((1, dout), lambda i: (0, 0)))
        args.append(bias.reshape(1, dout))
        fn = body
    else:
        fn = lambda x_ref, w_ref, o_ref: body(x_ref, w_ref, None, o_ref)

    return pl.pallas_call(
        fn,
        grid=(NP // RT,),
        in_specs=in_specs,
        out_specs=pl.BlockSpec((ncout, RT, dco), lambda i: (0, i, 0)),
        out_shape=jax.ShapeDtypeStruct((ncout, NP, dco), jnp.float32),
    )(*args)


def _head(h, fc1_w, fc1_b, fc2_w, fc2_b, fco_w, fco_b):
    def body(x_ref, w1_ref, b1_ref, w2_ref, b2_ref, wo_ref, bo_ref, o_ref):
        x = jnp.concatenate([x_ref[0], x_ref[1]], axis=1)
        a = jnp.maximum(_dotT(x, w1_ref[...]) + b1_ref[...], 0.0)
        a = jnp.maximum(_dotT(a, w2_ref[...]) + b2_ref[...], 0.0)
        lg = _dotT(a, wo_ref[...]) + bo_ref[...]
        m = jnp.max(lg, axis=1, keepdims=True)
        e = jnp.exp(lg - m)
        o_ref[...] = e / jnp.sum(e, axis=1, keepdims=True)

    return pl.pallas_call(
        body,
        grid=(NP // RT,),
        in_specs=[
            pl.BlockSpec((2, RT, 32), lambda i: (0, i, 0)),
            pl.BlockSpec((64, 64), lambda i: (0, 0)),
            pl.BlockSpec((1, 64), lambda i: (0, 0)),
            pl.BlockSpec((64, 64), lambda i: (0, 0)),
            pl.BlockSpec((1, 64), lambda i: (0, 0)),
            pl.BlockSpec((2, 64), lambda i: (0, 0)),
            pl.BlockSpec((1, 2), lambda i: (0, 0)),
        ],
        out_specs=pl.BlockSpec((RT, 2), lambda i: (i, 0)),
        out_shape=jax.ShapeDtypeStruct((NP, 2), jnp.float32),
    )(h, fc1_w, fc1_b.reshape(1, 64), fc2_w, fc2_b.reshape(1, 64),
      fco_w, fco_b.reshape(1, 2))


def kernel(x, hyperedge_index, W1, b1, W2, b2, W3, b3, W4, b4, W5, b5,
           fc1_w, fc1_b, fc2_w, fc2_b, fco_w, fco_b):
    src = hyperedge_index[0]
    dst = hyperedge_index[1]
    npad = EP - E
    pad_idx = (jnp.arange(npad, dtype=jnp.int32) % (NP - N)) + N
    srcp = jnp.concatenate([src, pad_idx]).reshape(16, NITER, CW)
    dstp = jnp.concatenate([dst, pad_idx]).reshape(16, NITER, CW)

    dinv, binv = _degree_kernel(srcp, dstp)

    xp = jnp.pad(x, ((0, NP - N), (0, 0)))
    xc = jnp.stack([xp[:, i * 32:(i + 1) * 32] for i in range(4)])
    dummy = jnp.zeros((1,), jnp.float32)
    m1 = _hconv128(xc, srcp, dstp, binv, dinv, dummy)
    h = _mm(m1, W1, b1, "elu", 8, 32)

    for W, b in ((W2, b2), (W3, b3), (W4, b4)):
        t = _mm(h, W, None, None, 8, 32)
        h = _hconv256(t, srcp, dstp, binv, dinv, b)

    t5 = _mm(h, W5, None, None, 2, 32)
    h5 = _hconv64(t5, srcp, dstp, binv, dinv, b5)

    out = _head(h5, fc1_w, fc1_b, fc2_w, fc2_b, fco_w, fco_b)
    return out[:N]


# speedup vs baseline: 9.7167x; 1.2072x over previous
import functools

import jax
import jax.numpy as jnp
from jax import lax
from jax.experimental import pallas as pl
from jax.experimental.pallas import tpu as pltpu
from jax.experimental.pallas import tpu_sc as plsc

N = 10000
NP = 10240
E = 320000
EP = 327680
PER_TILE = EP // 16
NITER = 160
CW = 128
RPT = NP // 16

_MESH = plsc.VectorSubcoreMesh(core_axis_name="c", subcore_axis_name="s")


def _zero_rows(buf, nrows, ncols16):
    z = jnp.zeros((16,), jnp.float32)

    def body(r, _):
        for j in range(ncols16):
            buf[r, pl.ds(j * 16, 16)] = z
        return 0

    lax.fori_loop(0, nrows, body, 0)


def _degree_kernel(srcp, dstp):
    def body(src_hbm, dst_hbm, dinv_hbm, binv_hbm, idxv, ones_v, buf, acc, sem):
        c = lax.axis_index("c")
        s = lax.axis_index("s")
        for i in range(CW // 16):
            ones_v[pl.ds(i * 16, 16)] = jnp.ones((16,), jnp.float32)

        def bzero(i, _):
            buf[pl.ds(i * 16, 16)] = jnp.zeros((16,), jnp.float32)
            return 0

        lax.fori_loop(0, RPT // 16, bzero, 0)
        pltpu.sync_copy(buf, acc.at[pl.ds(s * RPT, RPT)])

        @pl.when(c == 0)
        def _():
            pltpu.sync_copy(src_hbm.at[s], idxv)

        @pl.when(c == 1)
        def _():
            pltpu.sync_copy(dst_hbm.at[s], idxv)

        plsc.subcore_barrier()

        def scat(j, _):
            pltpu.sync_copy(ones_v, acc.at[idxv.at[j]], add=True)
            return 0

        lax.fori_loop(0, NITER, scat, 0)
        plsc.subcore_barrier()

        pltpu.sync_copy(acc.at[pl.ds(s * RPT, RPT)], buf)

        def inv(i, _):
            v = buf[pl.ds(i * 16, 16)]
            buf[pl.ds(i * 16, 16)] = jnp.where(v > 0.0, 1.0 / v, 0.0)
            return 0

        lax.fori_loop(0, RPT // 16, inv, 0)

        @pl.when(c == 0)
        def _():
            pltpu.sync_copy(buf, dinv_hbm.at[pl.ds(s * RPT, RPT)])

        @pl.when(c == 1)
        def _():
            pltpu.sync_copy(buf, binv_hbm.at[pl.ds(s * RPT, RPT)])

    f = pl.kernel(
        body,
        out_type=(
            jax.ShapeDtypeStruct((NP,), jnp.float32),
            jax.ShapeDtypeStruct((NP,), jnp.float32),
        ),
        mesh=_MESH,
        scratch_types=(
            pltpu.VMEM((NITER, CW), jnp.int32),
            pltpu.VMEM((CW,), jnp.float32),
            pltpu.VMEM((RPT,), jnp.float32),
            pltpu.VMEM_SHARED((NP,), jnp.float32),
            pltpu.SemaphoreType.DMA,
        ),
    )
    return f(srcp, dstp)


def _make_hconv(nchunks, dc, fuse_bias_elu):
    nc_per_sc = nchunks // 2
    c16 = dc // 16

    def body(f_hbm, src_hbm, dst_hbm, binv_hbm, dinv_hbm, bias_hbm, o_hbm,
             srcv, dstv, rows0, rows1, rows2, rows3, rbuf, svec, bvec,
             acc_e, acc_o, sem0, sem1, sem2, sem3):
        c = lax.axis_index("c")
        s = lax.axis_index("s")
        pltpu.sync_copy(src_hbm.at[s], srcv)
        pltpu.sync_copy(dst_hbm.at[s], dstv)
        bufs = (rows0, rows1, rows2, rows3)
        sems = (sem0, sem1, sem2, sem3)

        def pipelined_pass(gather_src, gidx, sidx, acc):
            def gref(j):
                return gather_src.at[gidx.at[j]]

            for b in range(3):
                pltpu.async_copy(gref(b), bufs[b], sems[b])

            def it(g, _):
                for b in range(4):
                    j = 4 * g + b
                    nb = (b + 3) % 4
                    pltpu.make_async_copy(gref(j), bufs[b], sems[b]).wait()

                    @pl.when(j + 3 < NITER)
                    def _():
                        pltpu.async_copy(gref(j + 3), bufs[nb], sems[nb])

                    pltpu.sync_copy(bufs[b], acc.at[sidx.at[j]], add=True)
                return 0

            lax.fori_loop(0, NITER // 4, it, 0)

        def do_chunk(k):
            _zero_rows(rbuf, RPT, c16)
            pltpu.sync_copy(rbuf, acc_e.at[pl.ds(s * RPT, RPT)])
            pltpu.sync_copy(rbuf, acc_o.at[pl.ds(s * RPT, RPT)])
            plsc.subcore_barrier()

            pipelined_pass(f_hbm.at[k], srcv, dstv, acc_e)
            plsc.subcore_barrier()

            pltpu.sync_copy(binv_hbm.at[pl.ds(s * RPT, RPT)], svec)
            pltpu.sync_copy(acc_e.at[pl.ds(s * RPT, RPT)], rbuf)

            def srow(g, _):
                sv = svec[pl.ds(g * 16, 16)]
                for rr in range(16):
                    b = sv[rr]
                    r = g * 16 + rr
                    for j in range(c16):
                        rbuf[r, pl.ds(j * 16, 16)] = (
                            rbuf[r, pl.ds(j * 16, 16)] * b)
                return 0

            lax.fori_loop(0, RPT // 16, srow, 0)
            pltpu.sync_copy(rbuf, acc_e.at[pl.ds(s * RPT, RPT)])
            plsc.subcore_barrier()

            pipelined_pass(acc_e, dstv, srcv, acc_o)
            plsc.subcore_barrier()

            pltpu.sync_copy(dinv_hbm.at[pl.ds(s * RPT, RPT)], svec)
            pltpu.sync_copy(acc_o.at[pl.ds(s * RPT, RPT)], rbuf)
            if fuse_bias_elu:
                pltpu.sync_copy(bias_hbm.at[pl.ds(k * dc, dc)], bvec)

            def erow(g, _):
                sv = svec[pl.ds(g * 16, 16)]
                for rr in range(16):
                    d = sv[rr]
                    r = g * 16 + rr
                    for j in range(c16):
                        v = rbuf[r, pl.ds(j * 16, 16)] * d
                        if fuse_bias_elu:
                            v = v + bvec[pl.ds(j * 16, 16)]
                            v = jnp.where(v > 0.0, v, jnp.exp(v) - 1.0)
                        rbuf[r, pl.ds(j * 16, 16)] = v
                return 0

            lax.fori_loop(0, RPT // 16, erow, 0)
            pltpu.sync_copy(rbuf, o_hbm.at[k].at[pl.ds(s * RPT, RPT)])
            plsc.subcore_barrier()

        @pl.when(c == 0)
        def _():
            for k in range(nc_per_sc):
                do_chunk(k)

        @pl.when(c == 1)
        def _():
            for k in range(nc_per_sc, nchunks):
                do_chunk(k)

    f = pl.kernel(
        body,
        out_type=jax.ShapeDtypeStruct((nchunks, NP, dc), jnp.float32),
        mesh=_MESH,
        scratch_types=(
            pltpu.VMEM((NITER, CW), jnp.int32),
            pltpu.VMEM((NITER, CW), jnp.int32),
            pltpu.VMEM((CW, dc), jnp.float32),
            pltpu.VMEM((CW, dc), jnp.float32),
            pltpu.VMEM((CW, dc), jnp.float32),
            pltpu.VMEM((CW, dc), jnp.float32),
            pltpu.VMEM((RPT, dc), jnp.float32),
            pltpu.VMEM((RPT,), jnp.float32),
            pltpu.VMEM((dc,), jnp.float32),
            pltpu.VMEM_SHARED((NP, dc), jnp.float32),
            pltpu.VMEM_SHARED((NP, dc), jnp.float32),
            pltpu.SemaphoreType.DMA,
            pltpu.SemaphoreType.DMA,
            pltpu.SemaphoreType.DMA,
            pltpu.SemaphoreType.DMA,
        ),
        compiler_params=pltpu.CompilerParams(use_tc_tiling_on_sc=False),
    )
    return f


_hconv128 = _make_hconv(4, 32, False)
_hconv256 = _make_hconv(8, 32, True)
_hconv64 = _make_hconv(2, 32, True)


def _dotT(x, w):
    return lax.dot_general(x, w, (((1,), (1,)), ((), ())),
                           preferred_element_type=jnp.float32)


RT = 1280


def _mm(xc, w, bias, act, ncout, dco):
    ncin, _, dci = xc.shape
    dout = w.shape[0]

    def body(x_ref, w_ref, b_ref, o_ref):
        wm = w_ref[...]
        acc = jnp.zeros((RT, dout), jnp.float32)
        for k in range(ncin):
            acc = acc + _dotT(x_ref[k], wm[:, k * dci:(k + 1) * dci])
        if b_ref is not None:
            acc = acc + b_ref[...]
        if act == "elu":
            acc = jnp.where(acc > 0.0, acc, jnp.exp(acc) - 1.0)
        for k in range(ncout):
            o_ref[k] = acc[:, k * dco:(k + 1) * dco]

    in_specs = [
        pl.BlockSpec((ncin, RT, dci), lambda i: (0, i, 0)),
        pl.BlockSpec((dout, w.shape[1]), lambda i: (0, 0)),
    ]
    args = [xc, w]
    if bias is not None:
        in_specs.append(pl.BlockSpec((1, dout), lambda i: (0, 0)))
        args.append(bias.reshape(1, dout))
        fn = body
    else:
        fn = lambda x_ref, w_ref, o_ref: body(x_ref, w_ref, None, o_ref)

    return pl.pallas_call(
        fn,
        grid=(NP // RT,),
        in_specs=in_specs,
        out_specs=pl.BlockSpec((ncout, RT, dco), lambda i: (0, i, 0)),
        out_shape=jax.ShapeDtypeStruct((ncout, NP, dco), jnp.float32),
    )(*args)


def _head(h, fc1_w, fc1_b, fc2_w, fc2_b, fco_w, fco_b):
    def body(x_ref, w1_ref, b1_ref, w2_ref, b2_ref, wo_ref, bo_ref, o_ref):
        x = jnp.concatenate([x_ref[0], x_ref[1]], axis=1)
        a = jnp.maximum(_dotT(x, w1_ref[...]) + b1_ref[...], 0.0)
        a = jnp.maximum(_dotT(a, w2_ref[...]) + b2_ref[...], 0.0)
        lg = _dotT(a, wo_ref[...]) + bo_ref[...]
        m = jnp.max(lg, axis=1, keepdims=True)
        e = jnp.exp(lg - m)
        o_ref[...] = e / jnp.sum(e, axis=1, keepdims=True)

    return pl.pallas_call(
        body,
        grid=(NP // RT,),
        in_specs=[
            pl.BlockSpec((2, RT, 32), lambda i: (0, i, 0)),
            pl.BlockSpec((64, 64), lambda i: (0, 0)),
            pl.BlockSpec((1, 64), lambda i: (0, 0)),
            pl.BlockSpec((64, 64), lambda i: (0, 0)),
            pl.BlockSpec((1, 64), lambda i: (0, 0)),
            pl.BlockSpec((2, 64), lambda i: (0, 0)),
            pl.BlockSpec((1, 2), lambda i: (0, 0)),
        ],
        out_specs=pl.BlockSpec((RT, 2), lambda i: (i, 0)),
        out_shape=jax.ShapeDtypeStruct((NP, 2), jnp.float32),
    )(h, fc1_w, fc1_b.reshape(1, 64), fc2_w, fc2_b.reshape(1, 64),
      fco_w, fco_b.reshape(1, 2))


def kernel(x, hyperedge_index, W1, b1, W2, b2, W3, b3, W4, b4, W5, b5,
           fc1_w, fc1_b, fc2_w, fc2_b, fco_w, fco_b):
    src = hyperedge_index[0]
    dst = hyperedge_index[1]
    npad = EP - E
    pad_idx = (jnp.arange(npad, dtype=jnp.int32) % (NP - N)) + N
    srcp = jnp.concatenate([src, pad_idx]).reshape(16, NITER, CW)
    dstp = jnp.concatenate([dst, pad_idx]).reshape(16, NITER, CW)

    dinv, binv = _degree_kernel(srcp, dstp)

    xp = jnp.pad(x, ((0, NP - N), (0, 0)))
    xc = jnp.stack([xp[:, i * 32:(i + 1) * 32] for i in range(4)])
    dummy = jnp.zeros((1,), jnp.float32)
    m1 = _hconv128(xc, srcp, dstp, binv, dinv, dummy)
    h = _mm(m1, W1, b1, "elu", 8, 32)

    for W, b in ((W2, b2), (W3, b3), (W4, b4)):
        t = _mm(h, W, None, None, 8, 32)
        h = _hconv256(t, srcp, dstp, binv, dinv, b)

    t5 = _mm(h, W5, None, None, 2, 32)
    h5 = _hconv64(t5, srcp, dstp, binv, dinv, b5)

    out = _head(h5, fc1_w, fc1_b, fc2_w, fc2_b, fco_w, fco_b)
    return out[:N]


# speedup vs baseline: 10.3168x; 1.0618x over previous
import functools

import jax
import jax.numpy as jnp
from jax import lax
from jax.experimental import pallas as pl
from jax.experimental.pallas import tpu as pltpu
from jax.experimental.pallas import tpu_sc as plsc

N = 10000
NP = 10240
E = 320000
EP = 327680
PER_TILE = EP // 16
NITER = 160
CW = 128
RPT = NP // 16

_MESH = plsc.VectorSubcoreMesh(core_axis_name="c", subcore_axis_name="s")


def _zero_rows(buf, nrows, ncols16):
    z = jnp.zeros((16,), jnp.float32)

    def body(r, _):
        for j in range(ncols16):
            buf[r, pl.ds(j * 16, 16)] = z
        return 0

    lax.fori_loop(0, nrows, body, 0)


def _degree_kernel(srcp, dstp):
    def body(src_hbm, dst_hbm, dinv_hbm, binv_hbm, idxv, ones_v, buf, acc, sem):
        c = lax.axis_index("c")
        s = lax.axis_index("s")
        for i in range(CW // 16):
            ones_v[pl.ds(i * 16, 16)] = jnp.ones((16,), jnp.float32)

        def bzero(i, _):
            buf[pl.ds(i * 16, 16)] = jnp.zeros((16,), jnp.float32)
            return 0

        lax.fori_loop(0, RPT // 16, bzero, 0)
        pltpu.sync_copy(buf, acc.at[pl.ds(s * RPT, RPT)])

        @pl.when(c == 0)
        def _():
            pltpu.sync_copy(src_hbm.at[s], idxv)

        @pl.when(c == 1)
        def _():
            pltpu.sync_copy(dst_hbm.at[s], idxv)

        plsc.subcore_barrier()

        def scat(j, _):
            pltpu.sync_copy(ones_v, acc.at[idxv.at[j]], add=True)
            return 0

        lax.fori_loop(0, NITER, scat, 0)
        plsc.subcore_barrier()

        pltpu.sync_copy(acc.at[pl.ds(s * RPT, RPT)], buf)

        def inv(i, _):
            v = buf[pl.ds(i * 16, 16)]
            buf[pl.ds(i * 16, 16)] = jnp.where(v > 0.0, 1.0 / v, 0.0)
            return 0

        lax.fori_loop(0, RPT // 16, inv, 0)

        @pl.when(c == 0)
        def _():
            pltpu.sync_copy(buf, dinv_hbm.at[pl.ds(s * RPT, RPT)])

        @pl.when(c == 1)
        def _():
            pltpu.sync_copy(buf, binv_hbm.at[pl.ds(s * RPT, RPT)])

    f = pl.kernel(
        body,
        out_type=(
            jax.ShapeDtypeStruct((NP,), jnp.float32),
            jax.ShapeDtypeStruct((NP,), jnp.float32),
        ),
        mesh=_MESH,
        scratch_types=(
            pltpu.VMEM((NITER, CW), jnp.int32),
            pltpu.VMEM((CW,), jnp.float32),
            pltpu.VMEM((RPT,), jnp.float32),
            pltpu.VMEM_SHARED((NP,), jnp.float32),
            pltpu.SemaphoreType.DMA,
        ),
    )
    return f(srcp, dstp)


def _make_hconv(nchunks, dc, fuse_bias_elu):
    nc_per_sc = nchunks // 2
    c16 = dc // 16

    def body(f_hbm, src_hbm, dst_hbm, binv_hbm, dinv_hbm, bias_hbm, o_hbm,
             srcv, dstv, rows0, rows1, rows2, rows3, rbuf, svec, bvec,
             acc_e, acc_o, sem0, sem1, sem2, sem3, sem4, sem5, sem6, sem7):
        c = lax.axis_index("c")
        s = lax.axis_index("s")
        pltpu.sync_copy(src_hbm.at[s], srcv)
        pltpu.sync_copy(dst_hbm.at[s], dstv)
        bufs = (rows0, rows1, rows2, rows3)
        gsems = (sem0, sem1, sem2, sem3)
        ssems = (sem4, sem5, sem6, sem7)

        def pipelined_pass(gather_src, gidx, sidx, acc):
            def gref(j):
                return gather_src.at[gidx.at[j]]

            def sref(j):
                return acc.at[sidx.at[j]]

            for b in range(3):
                pltpu.async_copy(gref(b), bufs[b], gsems[b])

            def it(g, _):
                for b in range(4):
                    j = 4 * g + b
                    nb = (b + 3) % 4
                    pltpu.make_async_copy(gref(j), bufs[b], gsems[b]).wait()
                    pltpu.async_copy(bufs[b], sref(j), ssems[b], add=True)

                    @pl.when(j + 3 < NITER)
                    def _():
                        @pl.when(j >= 1)
                        def _():
                            pltpu.make_async_copy(
                                bufs[nb], sref(j - 1), ssems[nb]).wait()

                        pltpu.async_copy(gref(j + 3), bufs[nb], gsems[nb])
                return 0

            lax.fori_loop(0, NITER // 4, it, 0)
            for b in range(4):
                pltpu.make_async_copy(
                    bufs[b], sref(NITER - 4 + b), ssems[b]).wait()

        def do_chunk(k):
            _zero_rows(rbuf, RPT, c16)
            pltpu.sync_copy(rbuf, acc_e.at[pl.ds(s * RPT, RPT)])
            pltpu.sync_copy(rbuf, acc_o.at[pl.ds(s * RPT, RPT)])
            plsc.subcore_barrier()

            pipelined_pass(f_hbm.at[k], srcv, dstv, acc_e)
            plsc.subcore_barrier()

            pltpu.sync_copy(binv_hbm.at[pl.ds(s * RPT, RPT)], svec)
            pltpu.sync_copy(acc_e.at[pl.ds(s * RPT, RPT)], rbuf)

            def srow(g, _):
                sv = svec[pl.ds(g * 16, 16)]
                for rr in range(16):
                    b = sv[rr]
                    r = g * 16 + rr
                    for j in range(c16):
                        rbuf[r, pl.ds(j * 16, 16)] = (
                            rbuf[r, pl.ds(j * 16, 16)] * b)
                return 0

            lax.fori_loop(0, RPT // 16, srow, 0)
            pltpu.sync_copy(rbuf, acc_e.at[pl.ds(s * RPT, RPT)])
            plsc.subcore_barrier()

            pipelined_pass(acc_e, dstv, srcv, acc_o)
            plsc.subcore_barrier()

            pltpu.sync_copy(dinv_hbm.at[pl.ds(s * RPT, RPT)], svec)
            pltpu.sync_copy(acc_o.at[pl.ds(s * RPT, RPT)], rbuf)
            if fuse_bias_elu:
                pltpu.sync_copy(bias_hbm.at[pl.ds(k * dc, dc)], bvec)

            def erow(g, _):
                sv = svec[pl.ds(g * 16, 16)]
                for rr in range(16):
                    d = sv[rr]
                    r = g * 16 + rr
                    for j in range(c16):
                        v = rbuf[r, pl.ds(j * 16, 16)] * d
                        if fuse_bias_elu:
                            v = v + bvec[pl.ds(j * 16, 16)]
                            v = jnp.where(v > 0.0, v, jnp.exp(v) - 1.0)
                        rbuf[r, pl.ds(j * 16, 16)] = v
                return 0

            lax.fori_loop(0, RPT // 16, erow, 0)
            pltpu.sync_copy(rbuf, o_hbm.at[k].at[pl.ds(s * RPT, RPT)])
            plsc.subcore_barrier()

        @pl.when(c == 0)
        def _():
            for k in range(nc_per_sc):
                do_chunk(k)

        @pl.when(c == 1)
        def _():
            for k in range(nc_per_sc, nchunks):
                do_chunk(k)

    f = pl.kernel(
        body,
        out_type=jax.ShapeDtypeStruct((nchunks, NP, dc), jnp.float32),
        mesh=_MESH,
        scratch_types=(
            pltpu.VMEM((NITER, CW), jnp.int32),
            pltpu.VMEM((NITER, CW), jnp.int32),
            pltpu.VMEM((CW, dc), jnp.float32),
            pltpu.VMEM((CW, dc), jnp.float32),
            pltpu.VMEM((CW, dc), jnp.float32),
            pltpu.VMEM((CW, dc), jnp.float32),
            pltpu.VMEM((RPT, dc), jnp.float32),
            pltpu.VMEM((RPT,), jnp.float32),
            pltpu.VMEM((dc,), jnp.float32),
            pltpu.VMEM_SHARED((NP, dc), jnp.float32),
            pltpu.VMEM_SHARED((NP, dc), jnp.float32),
            pltpu.SemaphoreType.DMA,
            pltpu.SemaphoreType.DMA,
            pltpu.SemaphoreType.DMA,
            pltpu.SemaphoreType.DMA,
            pltpu.SemaphoreType.DMA,
            pltpu.SemaphoreType.DMA,
            pltpu.SemaphoreType.DMA,
            pltpu.SemaphoreType.DMA,
        ),
        compiler_params=pltpu.CompilerParams(use_tc_tiling_on_sc=False),
    )
    return f


_hconv128 = _make_hconv(4, 32, False)
_hconv256 = _make_hconv(8, 32, True)
_hconv64 = _make_hconv(2, 32, True)


def _dotT(x, w):
    return lax.dot_general(x, w, (((1,), (1,)), ((), ())),
                           preferred_element_type=jnp.float32)


RT = 1280


def _mm(xc, w, bias, act, ncout, dco):
    ncin, _, dci = xc.shape
    dout = w.shape[0]

    def body(x_ref, w_ref, b_ref, o_ref):
        wm = w_ref[...]
        acc = jnp.zeros((RT, dout), jnp.float32)
        for k in range(ncin):
            acc = acc + _dotT(x_ref[k], wm[:, k * dci:(k + 1) * dci])
        if b_ref is not None:
            acc = acc + b_ref[...]
        if act == "elu":
            acc = jnp.where(acc > 0.0, acc, jnp.exp(acc) - 1.0)
        for k in range(ncout):
            o_ref[k] = acc[:, k * dco:(k + 1) * dco]

    in_specs = [
        pl.BlockSpec((ncin, RT, dci), lambda i: (0, i, 0)),
        pl.BlockSpec((dout, w.shape[1]), lambda i: (0, 0)),
    ]
    args = [xc, w]
    if bias is not None:
        in_specs.append(pl.BlockSpec((1, dout), lambda i: (0, 0)))
        args.append(bias.reshape(1, dout))
        fn = body
    else:
        fn = lambda x_ref, w_ref, o_ref: body(x_ref, w_ref, None, o_ref)

    return pl.pallas_call(
        fn,
        grid=(NP // RT,),
        in_specs=in_specs,
        out_specs=pl.BlockSpec((ncout, RT, dco), lambda i: (0, i, 0)),
        out_shape=jax.ShapeDtypeStruct((ncout, NP, dco), jnp.float32),
    )(*args)


def _head(h, fc1_w, fc1_b, fc2_w, fc2_b, fco_w, fco_b):
    def body(x_ref, w1_ref, b1_ref, w2_ref, b2_ref, wo_ref, bo_ref, o_ref):
        x = jnp.concatenate([x_ref[0], x_ref[1]], axis=1)
        a = jnp.maximum(_dotT(x, w1_ref[...]) + b1_ref[...], 0.0)
        a = jnp.maximum(_dotT(a, w2_ref[...]) + b2_ref[...], 0.0)
        lg = _dotT(a, wo_ref[...]) + bo_ref[...]
        m = jnp.max(lg, axis=1, keepdims=True)
        e = jnp.exp(lg - m)
        o_ref[...] = e / jnp.sum(e, axis=1, keepdims=True)

    return pl.pallas_call(
        body,
        grid=(NP // RT,),
        in_specs=[
            pl.BlockSpec((2, RT, 32), lambda i: (0, i, 0)),
            pl.BlockSpec((64, 64), lambda i: (0, 0)),
            pl.BlockSpec((1, 64), lambda i: (0, 0)),
            pl.BlockSpec((64, 64), lambda i: (0, 0)),
            pl.BlockSpec((1, 64), lambda i: (0, 0)),
            pl.BlockSpec((2, 64), lambda i: (0, 0)),
            pl.BlockSpec((1, 2), lambda i: (0, 0)),
        ],
        out_specs=pl.BlockSpec((RT, 2), lambda i: (i, 0)),
        out_shape=jax.ShapeDtypeStruct((NP, 2), jnp.float32),
    )(h, fc1_w, fc1_b.reshape(1, 64), fc2_w, fc2_b.reshape(1, 64),
      fco_w, fco_b.reshape(1, 2))


def kernel(x, hyperedge_index, W1, b1, W2, b2, W3, b3, W4, b4, W5, b5,
           fc1_w, fc1_b, fc2_w, fc2_b, fco_w, fco_b):
    src = hyperedge_index[0]
    dst = hyperedge_index[1]
    npad = EP - E
    pad_idx = (jnp.arange(npad, dtype=jnp.int32) % (NP - N)) + N
    srcp = jnp.concatenate([src, pad_idx]).reshape(16, NITER, CW)
    dstp = jnp.concatenate([dst, pad_idx]).reshape(16, NITER, CW)

    dinv, binv = _degree_kernel(srcp, dstp)

    xp = jnp.pad(x, ((0, NP - N), (0, 0)))
    xc = jnp.stack([xp[:, i * 32:(i + 1) * 32] for i in range(4)])
    dummy = jnp.zeros((1,), jnp.float32)
    m1 = _hconv128(xc, srcp, dstp, binv, dinv, dummy)
    h = _mm(m1, W1, b1, "elu", 8, 32)

    for W, b in ((W2, b2), (W3, b3), (W4, b4)):
        t = _mm(h, W, None, None, 8, 32)
        h = _hconv256(t, srcp, dstp, binv, dinv, b)

    t5 = _mm(h, W5, None, None, 2, 32)
    h5 = _hconv64(t5, srcp, dstp, binv, dinv, b5)

    out = _head(h5, fc1_w, fc1_b, fc2_w, fc2_b, fco_w, fco_b)
    return out[:N]
